# baseline (speedup 1.0000x reference)
import numpy as np
import jax
import jax.numpy as jnp
from jax import lax
from jax.experimental import pallas as pl
from jax.experimental.pallas import tpu as pltpu
from jax.experimental.pallas import tpu_sc as plsc

_B = 4096
_F = 256
_M = 128
_NPAIR = _M * (_M + 1) // 2
_OUT = _M + _NPAIR
_NW = 32
_RPW = _B // _NW
_L = 16
_NWIN = _OUT // _L
_G = 8
_NG = _RPW // _G
_LHALF = 264
_CL = _LHALF * _L
_CR = _OUT - _CL
_XSTR = 160
_RB = 16


def _seg_off(i):
    return i * _M - i * (i - 1) // 2


def _seg_table():
    seg = [0] * _NPAIR
    for i in range(_M):
        for p in range(_seg_off(i), _seg_off(i) + _M - i):
            seg[p] = i
    return seg


_SEG = _seg_table()


def _window_plan():
    plan = {}
    gat_ia, gat_ib = [], []
    for w in range(8, _NWIN):
        p0 = 16 * w - _M
        segs = sorted(set(_SEG[p0:p0 + _L]))
        if len(segs) == 1:
            i = segs[0]
            plan[w] = ("int", i, i + (p0 - _seg_off(i)))
        elif len(segs) == 2:
            i, i2 = segs
            plan[w] = ("sel", i, _seg_off(i2) - p0,
                       i + (p0 - _seg_off(i)), i2 + (p0 - _seg_off(i2)))
        else:
            k = len(gat_ia)
            ia = [_SEG[p0 + l] for l in range(_L)]
            ib = [_SEG[p0 + l] + (p0 + l - _seg_off(_SEG[p0 + l])) for l in range(_L)]
            gat_ia.append(ia)
            gat_ib.append(ib)
            plan[w] = ("gat", k)
    return plan, np.asarray(gat_ia, np.int32).reshape(-1), np.asarray(gat_ib, np.int32).reshape(-1)


_PLAN, _GAT_IA, _GAT_IB = _window_plan()


def _emit_half(s, xmb8, ga_v, gb_v, buf, w_lo, w_hi, col0):
    rb = _RB + s * _XSTR
    rbv = jnp.full((_L,), rb, jnp.int32)
    lane = lax.iota(jnp.int32, _L)
    xv = [xmb8[pl.ds(rb + _L * c, _L)] for c in range(_M // _L)]
    splats = {}

    def splat(i):
        if i not in splats:
            splats[i] = jnp.full((_L,), xv[i // _L][i % _L])
        return splats[i]

    for w in range(w_lo, w_hi):
        c = _L * w - col0
        if w < 8:
            buf[s, pl.ds(c, _L)] = xv[w]
            continue
        kind = _PLAN[w]
        if kind[0] == "int":
            _, i, j0 = kind
            buf[s, pl.ds(c, _L)] = splat(i) * xmb8[pl.ds(rb + j0, _L)]
        elif kind[0] == "sel":
            _, i, t, j0a, j0b = kind
            va = xmb8[pl.ds(rb + j0a, _L)]
            vb = xmb8[pl.ds(rb + j0b, _L)]
            buf[s, pl.ds(c, _L)] = jnp.where(lane < t, splat(i) * va, splat(i + 1) * vb)
        else:
            _, k = kind
            va = plsc.load_gather(xmb8, [rbv + ga_v[pl.ds(_L * k, _L)]])
            vb = plsc.load_gather(xmb8, [rbv + gb_v[pl.ds(_L * k, _L)]])
            buf[s, pl.ds(c, _L)] = va * vb


def _body(x_hbm, m_hbm, ga_hbm, gb_hbm, out_hbm,
          xblk, mi_v, ga_v, gb_v, xmb8, bufl, bufr, seml, semr):
    wid = lax.axis_index("s") * 2 + lax.axis_index("c")
    base = wid * _RPW
    pltpu.sync_copy(x_hbm.at[pl.ds(base * _F, _RPW * _F)], xblk)
    pltpu.sync_copy(m_hbm, mi_v)
    pltpu.sync_copy(ga_hbm, ga_v)
    pltpu.sync_copy(gb_hbm, gb_v)

    def _dstl(g):
        return out_hbm.at[pl.ds(base + _G * g, _G), pl.ds(0, _CL)]

    def _dstr(g):
        return out_hbm.at[pl.ds(base + _G * g, _G), pl.ds(_CL, _CR)]

    @pl.loop(0, _NG)
    def _group(g):
        @pl.loop(0, _G)
        def _stage(s):
            rb = _RB + s * _XSTR
            roffv = jnp.full((_L,), (g * _G + s) * _F, jnp.int32)
            for c in range(_M // _L):
                mi = mi_v[pl.ds(_L * c, _L)]
                xmb8[pl.ds(rb + _L * c, _L)] = plsc.load_gather(xblk, [roffv + mi])

        @pl.when(g > 0)
        def _():
            pltpu.make_async_copy(bufl, _dstl(g), seml).wait()

        @pl.loop(0, _G)
        def _rowl(s):
            _emit_half(s, xmb8, ga_v, gb_v, bufl, 0, _LHALF, 0)

        pltpu.async_copy(bufl, _dstl(g), seml)

        @pl.when(g > 0)
        def _():
            pltpu.make_async_copy(bufr, _dstr(g), semr).wait()

        @pl.loop(0, _G)
        def _rowr(s):
            _emit_half(s, xmb8, ga_v, gb_v, bufr, _LHALF, _NWIN, _CL)

        pltpu.async_copy(bufr, _dstr(g), semr)

    pltpu.make_async_copy(bufl, _dstl(_NG - 1), seml).wait()
    pltpu.make_async_copy(bufr, _dstr(_NG - 1), semr).wait()


def _mask_de(xflat, midx, gat_ia, gat_ib):
    f = pl.kernel(
        _body,
        out_type=jax.ShapeDtypeStruct((_B, _OUT), jnp.float32),
        mesh=plsc.VectorSubcoreMesh(core_axis_name="c", subcore_axis_name="s",
                                    num_cores=2, num_subcores=16),
        compiler_params=pltpu.CompilerParams(needs_layout_passes=False),
        scratch_types=[
            pltpu.VMEM((_RPW * _F,), jnp.float32),
            pltpu.VMEM((_M,), jnp.int32),
            pltpu.VMEM((_GAT_IA.size,), jnp.int32),
            pltpu.VMEM((_GAT_IB.size,), jnp.int32),
            pltpu.VMEM((_G * _XSTR,), jnp.float32),
            pltpu.VMEM((_G, _CL), jnp.float32),
            pltpu.VMEM((_G, _CR), jnp.float32),
            pltpu.SemaphoreType.DMA,
            pltpu.SemaphoreType.DMA,
        ],
    )
    return f(xflat, midx, gat_ia, gat_ib)


def kernel(x, mask):
    midx = jnp.argsort(~mask)[:_M].astype(jnp.int32)
    return _mask_de(x.reshape(-1), midx, jnp.asarray(_GAT_IA), jnp.asarray(_GAT_IB))


# speedup vs baseline: 2.3633x; 2.3633x over previous
import numpy as np
import jax
import jax.numpy as jnp
from jax import lax
from jax.experimental import pallas as pl
from jax.experimental.pallas import tpu as pltpu
from jax.experimental.pallas import tpu_sc as plsc

_B = 4096
_F = 256
_M = 128
_NPAIR = _M * (_M + 1) // 2
_OUT = _M + _NPAIR
_NW = 32
_RPW = _B // _NW
_L = 16
_NWIN = _OUT // _L
_G = 8
_NG = _RPW // _G
_LHALF = 264
_CL = _LHALF * _L
_CR = _OUT - _CL
_XSTR = 160
_RB = 16


def _seg_off(i):
    return i * _M - i * (i - 1) // 2


def _seg_table():
    seg = [0] * _NPAIR
    for i in range(_M):
        for p in range(_seg_off(i), _seg_off(i) + _M - i):
            seg[p] = i
    return seg


_SEG = _seg_table()


def _window_plan():
    plan = {}
    gat_ia, gat_ib = [], []
    for w in range(8, _NWIN):
        p0 = 16 * w - _M
        segs = sorted(set(_SEG[p0:p0 + _L]))
        if len(segs) == 1:
            i = segs[0]
            plan[w] = ("int", i, i + (p0 - _seg_off(i)))
        elif len(segs) == 2:
            i, i2 = segs
            plan[w] = ("sel", i, _seg_off(i2) - p0,
                       i + (p0 - _seg_off(i)), i2 + (p0 - _seg_off(i2)))
        else:
            k = len(gat_ia)
            ia = [_SEG[p0 + l] for l in range(_L)]
            ib = [_SEG[p0 + l] + (p0 + l - _seg_off(_SEG[p0 + l])) for l in range(_L)]
            gat_ia.append(ia)
            gat_ib.append(ib)
            plan[w] = ("gat", k)
    return plan, np.asarray(gat_ia, np.int32).reshape(-1), np.asarray(gat_ib, np.int32).reshape(-1)


_PLAN, _GAT_IA, _GAT_IB = _window_plan()


_DEPTH = 10


def _emit_half(s, xmb8, ga_v, gb_v, buf, w_lo, w_hi, col0):
    rb = _RB + s * _XSTR
    rbv = jnp.full((_L,), rb, jnp.int32)
    lane = lax.iota(jnp.int32, _L)
    xv = [xmb8[pl.ds(rb + _L * c, _L)] for c in range(_M // _L)]
    splats = {}

    def splat(i):
        if i not in splats:
            splats[i] = jnp.full((_L,), xv[i // _L][i % _L])
        return splats[i]

    pend = {}

    def issue(w):
        if w < 8:
            pend[w] = ()
            return
        kind = _PLAN[w]
        if kind[0] == "int":
            pend[w] = (xmb8[pl.ds(rb + kind[2], _L)],)
        elif kind[0] == "sel":
            pend[w] = (xmb8[pl.ds(rb + kind[3], _L)],
                       xmb8[pl.ds(rb + kind[4], _L)])
        else:
            k = kind[1]
            va = plsc.load_gather(xmb8, [rbv + ga_v[pl.ds(_L * k, _L)]])
            vb = plsc.load_gather(xmb8, [rbv + gb_v[pl.ds(_L * k, _L)]])
            pend[w] = (va, vb)

    def finish(w):
        c = _L * w - col0
        vals = pend.pop(w)
        if w < 8:
            buf[s, pl.ds(c, _L)] = xv[w]
            return
        kind = _PLAN[w]
        if kind[0] == "int":
            buf[s, pl.ds(c, _L)] = splat(kind[1]) * vals[0]
        elif kind[0] == "sel":
            _, i, t, _, _ = kind
            buf[s, pl.ds(c, _L)] = jnp.where(
                lane < t, splat(i) * vals[0], splat(i + 1) * vals[1])
        else:
            buf[s, pl.ds(c, _L)] = vals[0] * vals[1]

    ws = list(range(w_lo, w_hi))
    for t in range(len(ws) + _DEPTH):
        if t < len(ws):
            issue(ws[t])
        if t >= _DEPTH:
            finish(ws[t - _DEPTH])


def _body(x_hbm, m_hbm, ga_hbm, gb_hbm, out_hbm,
          xblk, mi_v, ga_v, gb_v, xmb8, bufl, bufr, seml, semr):
    wid = lax.axis_index("s") * 2 + lax.axis_index("c")
    base = wid * _RPW
    pltpu.sync_copy(x_hbm.at[pl.ds(base * _F, _RPW * _F)], xblk)
    pltpu.sync_copy(m_hbm, mi_v)
    pltpu.sync_copy(ga_hbm, ga_v)
    pltpu.sync_copy(gb_hbm, gb_v)

    def _dstl(g):
        return out_hbm.at[pl.ds(base + _G * g, _G), pl.ds(0, _CL)]

    def _dstr(g):
        return out_hbm.at[pl.ds(base + _G * g, _G), pl.ds(_CL, _CR)]

    @pl.loop(0, _NG)
    def _group(g):
        @pl.loop(0, _G)
        def _stage(s):
            rb = _RB + s * _XSTR
            roffv = jnp.full((_L,), (g * _G + s) * _F, jnp.int32)
            idxs = [roffv + mi_v[pl.ds(_L * c, _L)] for c in range(_M // _L)]
            vals = [plsc.load_gather(xblk, [ix]) for ix in idxs]
            for c in range(_M // _L):
                xmb8[pl.ds(rb + _L * c, _L)] = vals[c]

        @pl.when(g > 0)
        def _():
            pltpu.make_async_copy(bufl, _dstl(g), seml).wait()

        @pl.loop(0, _G)
        def _rowl(s):
            _emit_half(s, xmb8, ga_v, gb_v, bufl, 0, _LHALF, 0)

        pltpu.async_copy(bufl, _dstl(g), seml)

        @pl.when(g > 0)
        def _():
            pltpu.make_async_copy(bufr, _dstr(g), semr).wait()

        @pl.loop(0, _G)
        def _rowr(s):
            _emit_half(s, xmb8, ga_v, gb_v, bufr, _LHALF, _NWIN, _CL)

        pltpu.async_copy(bufr, _dstr(g), semr)

    pltpu.make_async_copy(bufl, _dstl(_NG - 1), seml).wait()
    pltpu.make_async_copy(bufr, _dstr(_NG - 1), semr).wait()


def _mask_de(xflat, midx, gat_ia, gat_ib):
    f = pl.kernel(
        _body,
        out_type=jax.ShapeDtypeStruct((_B, _OUT), jnp.float32),
        mesh=plsc.VectorSubcoreMesh(core_axis_name="c", subcore_axis_name="s",
                                    num_cores=2, num_subcores=16),
        compiler_params=pltpu.CompilerParams(needs_layout_passes=False),
        scratch_types=[
            pltpu.VMEM((_RPW * _F,), jnp.float32),
            pltpu.VMEM((_M,), jnp.int32),
            pltpu.VMEM((_GAT_IA.size,), jnp.int32),
            pltpu.VMEM((_GAT_IB.size,), jnp.int32),
            pltpu.VMEM((_G * _XSTR,), jnp.float32),
            pltpu.VMEM((_G, _CL), jnp.float32),
            pltpu.VMEM((_G, _CR), jnp.float32),
            pltpu.SemaphoreType.DMA,
            pltpu.SemaphoreType.DMA,
        ],
    )
    return f(xflat, midx, gat_ia, gat_ib)


def kernel(x, mask):
    midx = jnp.argsort(~mask)[:_M].astype(jnp.int32)
    return _mask_de(x.reshape(-1), midx, jnp.asarray(_GAT_IA), jnp.asarray(_GAT_IB))


# speedup vs baseline: 3.6825x; 1.5582x over previous
import numpy as np
import jax
import jax.numpy as jnp
from jax import lax
from jax.experimental import pallas as pl
from jax.experimental.pallas import tpu as pltpu
from jax.experimental.pallas import tpu_sc as plsc

_B = 4096
_F = 256
_M = 128
_NPAIR = _M * (_M + 1) // 2
_OUT = _M + _NPAIR
_NW = 32
_RPW = _B // _NW
_L = 16
_NLG = _RPW // _L
_CHW = 64
_NCH = _OUT // _CHW
_XT = (_M + 1) * _RPW + 4 * _RPW


def _part_tables():
    off = lambda i: i * _M - i * (i - 1) // 2
    seg = np.zeros(_NPAIR, np.int32)
    for i in range(_M):
        seg[off(i):off(i) + _M - i] = i
    ii = np.empty(_OUT, np.int32)
    jj = np.empty(_OUT, np.int32)
    ii[:_M] = _M
    jj[:_M] = np.arange(_M)
    for c in range(_M, _OUT):
        p = c - _M
        i = int(seg[p])
        ii[c] = i
        jj[c] = i + (p - off(i))
    c0l, n, ia, jb = [], [], [], []
    for c in range(_OUT):
        if (c == 0 or ii[c] != ii[c - 1] or jj[c] != jj[c - 1] + 1
                or c % _CHW == 0):
            c0l.append(c % _CHW)
            n.append(0)
            ia.append(int(ii[c]) * _RPW)
            jb.append(int(jj[c]) * _RPW)
        n[-1] += 1
    starts = np.cumsum([0] + n[:-1])
    cpi = np.searchsorted(starts, np.arange(_NCH) * _CHW, side="left")
    cpi = np.append(cpi, len(n)).astype(np.int32)
    assert sum(n) == _OUT and max(n) <= _CHW
    return (np.asarray(c0l, np.int32), np.asarray(n, np.int32),
            np.asarray(ia, np.int32), np.asarray(jb, np.int32), cpi)


_C0L, _N, _IA, _JB, _CPI = _part_tables()
_NP = _C0L.size
_TBL = np.concatenate([_C0L, _N, _IA, _JB, _CPI])
_NTW = ((_TBL.size + 15) // 16) * 16


def _stage_smem(tbl_v, tbl_s):
    for t in range(_NTW // _L):
        v = tbl_v[pl.ds(_L * t, _L)]
        for k in range(_L):
            if _L * t + k < _TBL.size:
                tbl_s[_L * t + k] = v[k]


def _stage_xmt(xblk, mi_v, xmt):
    ones = jnp.full((_L,), 1.0, jnp.float32)
    for lg in range(_NLG):
        xmt[pl.ds(_M * _RPW + _L * lg, _L)] = ones
    iota256 = lax.iota(jnp.int32, _L) * _F
    bases = [iota256 + (lg * _L * _F) for lg in range(_NLG)]

    @pl.loop(0, _M // _L)
    def _f16(f16):
        mi = mi_v[pl.ds(_L * f16, _L)]
        base = f16 * (_L * _RPW)
        pend = {}
        for k in range(_L + 1):
            if k < _L:
                mib = jnp.full((_L,), mi[k])
                pend[k] = [plsc.load_gather(xblk, [bases[lg] + mib])
                           for lg in range(_NLG)]
            if k >= 1:
                vals = pend.pop(k - 1)
                for lg in range(_NLG):
                    xmt[pl.ds(base + _RPW * (k - 1) + _L * lg, _L)] = vals[lg]


def _chunk_body(ch, wait_pred, xmt, tbl_s, buf, sem, out_hbm, colbase):
    def _dst(t):
        return out_hbm.at[pl.ds(ch * _CHW + 8 * t, 8), pl.ds(colbase, _RPW)]

    def _waits():
        for t in range(8):
            pltpu.make_async_copy(buf.at[pl.ds(8 * t, 8), :], _dst(t), sem).wait()

    if wait_pred is None:
        _waits()
    else:
        pl.when(wait_pred)(_waits)

    plo = tbl_s[4 * _NP + ch]
    phi = tbl_s[4 * _NP + ch + 1]

    @pl.loop(plo, phi)
    def _part(pp):
        c0 = tbl_s[pp]
        npart = tbl_s[_NP + pp]
        ia = tbl_s[2 * _NP + pp]
        jb = tbl_s[3 * _NP + pp]
        va = [xmt[pl.ds(ia + _L * lg, _L)] for lg in range(_NLG)]

        @pl.loop(0, (npart + 3) >> 2)
        def _colq(kq):
            k0 = kq * 4
            vb, prod = {}, {}
            for u in range(4):
                jaddr = jb + (k0 + u) * _RPW
                vb[u] = [xmt[pl.ds(jaddr + _L * lg, _L)] for lg in range(_NLG)]
            for u in range(4):
                prod[u] = [va[lg] * vb[u][lg] for lg in range(_NLG)]
            for u in range(4):
                for lg in range(_NLG):
                    buf[c0 + k0 + u, pl.ds(_L * lg, _L)] = prod[u][lg]

    for t in range(8):
        pltpu.async_copy(buf.at[pl.ds(8 * t, 8), :], _dst(t), sem)


def _body(x_hbm, m_hbm, tbl_hbm, out_hbm, xblk, mi_v, tbl_v, xmt, bufa, bufb,
          tbl_s, sema, semb):
    wid = lax.axis_index("s") * 2 + lax.axis_index("c")
    base = wid * _RPW
    pltpu.sync_copy(x_hbm.at[pl.ds(base * _F, _RPW * _F)], xblk)
    pltpu.sync_copy(m_hbm, mi_v)
    pltpu.sync_copy(tbl_hbm, tbl_v)
    _stage_smem(tbl_v, tbl_s)
    _stage_xmt(xblk, mi_v, xmt)

    @pl.loop(0, _NCH - 1, step=2)
    def _chunks(ch):
        _chunk_body(ch, ch > 0, xmt, tbl_s, bufa, sema, out_hbm, base)
        _chunk_body(ch + 1, ch > 0, xmt, tbl_s, bufb, semb, out_hbm, base)

    _chunk_body(_NCH - 1, None, xmt, tbl_s, bufa, sema, out_hbm, base)

    def _dst(c, t):
        return out_hbm.at[pl.ds(c * _CHW + 8 * t, 8), pl.ds(base, _RPW)]

    for t in range(8):
        pltpu.make_async_copy(bufa.at[pl.ds(8 * t, 8), :], _dst(_NCH - 1, t), sema).wait()
        pltpu.make_async_copy(bufb.at[pl.ds(8 * t, 8), :], _dst(_NCH - 2, t), semb).wait()


def _mask_de(xflat, midx, tbl):
    f = pl.kernel(
        _body,
        out_type=jax.ShapeDtypeStruct((_OUT, _B), jnp.float32),
        mesh=plsc.VectorSubcoreMesh(core_axis_name="c", subcore_axis_name="s",
                                    num_cores=2, num_subcores=16),
        compiler_params=pltpu.CompilerParams(needs_layout_passes=False),
        scratch_types=[
            pltpu.VMEM((_RPW * _F,), jnp.float32),
            pltpu.VMEM((_M,), jnp.int32),
            pltpu.VMEM((_NTW,), jnp.int32),
            pltpu.VMEM((_XT,), jnp.float32),
            pltpu.VMEM((_CHW + 8, _RPW), jnp.float32),
            pltpu.VMEM((_CHW + 8, _RPW), jnp.float32),
            pltpu.SMEM((_TBL.size,), jnp.int32),
            pltpu.SemaphoreType.DMA,
            pltpu.SemaphoreType.DMA,
        ],
    )
    return f(xflat, midx, tbl)


def kernel(x, mask):
    midx = jnp.argsort(~mask)[:_M].astype(jnp.int32)
    tbl = jnp.asarray(np.pad(_TBL, (0, _NTW - _TBL.size)))
    out = _mask_de(x.reshape(-1), midx, tbl)
    return out.T
